# Initial kernel scaffold; baseline (speedup 1.0000x reference)
import jax
import jax.numpy as jnp
from jax.experimental import pallas as pl


def kernel(users, movies, Uw, Mw, W1, b1, W2, b2, W3, b3, Wf, bf):
    raise NotImplementedError("write your pallas kernel here")


# speedup vs baseline: 1.4281x; 1.4281x over previous
import functools

import jax
import jax.numpy as jnp
from jax import lax
from jax.experimental import pallas as pl
from jax.experimental.pallas import tpu as pltpu
from jax.experimental.pallas import tpu_sc as plsc

B = 16384
D = 50
NC = 2
NS = 16
NW = NC * NS
BPW = B // NW
ICHUNK = 128
NCHUNK = BPW // ICHUNK


def _gather_body(users_hbm, movies_hbm, uw_hbm, mw_hbm, uout_hbm, mout_hbm,
                 uidx, midx, urows, mrows, usem, msem):
    wid = lax.axis_index("s") * NC + lax.axis_index("c")
    base = wid * BPW
    pltpu.sync_copy(users_hbm.at[pl.ds(base, BPW)], uidx)
    pltpu.sync_copy(movies_hbm.at[pl.ds(base, BPW)], midx)

    L = 16
    for c in range(NCHUNK):
        off = c * ICHUNK

        def issue(g, _):
            uvec = uidx[pl.ds(off + g * L, L)]
            mvec = midx[pl.ds(off + g * L, L)]
            for l in range(L):
                pltpu.async_copy(uw_hbm.at[uvec[l]], urows.at[g * L + l], usem)
                pltpu.async_copy(mw_hbm.at[mvec[l]], mrows.at[g * L + l], msem)
            return 0

        lax.fori_loop(0, ICHUNK // L, issue, 0)

        def drain(i, _):
            pltpu.make_async_copy(uw_hbm.at[0], urows.at[0], usem).wait()
            pltpu.make_async_copy(mw_hbm.at[0], mrows.at[0], msem).wait()
            return 0

        lax.fori_loop(0, ICHUNK, drain, 0)
        pltpu.sync_copy(urows, uout_hbm.at[pl.ds(base + off, ICHUNK)])
        pltpu.sync_copy(mrows, mout_hbm.at[pl.ds(base + off, ICHUNK)])


@functools.cache
def _make_gather():
    return pl.kernel(
        _gather_body,
        out_type=(jax.ShapeDtypeStruct((B, D), jnp.float32),
                  jax.ShapeDtypeStruct((B, D), jnp.float32)),
        mesh=plsc.VectorSubcoreMesh(core_axis_name="c", subcore_axis_name="s",
                                    num_cores=NC, num_subcores=NS),
        scratch_types=[
            pltpu.VMEM((BPW,), jnp.int32),
            pltpu.VMEM((BPW,), jnp.int32),
            pltpu.VMEM((ICHUNK, D), jnp.float32),
            pltpu.VMEM((ICHUNK, D), jnp.float32),
            pltpu.SemaphoreType.DMA,
            pltpu.SemaphoreType.DMA,
        ],
    )


BB = 2048


def _split(w):
    hi = w.astype(jnp.bfloat16)
    lo = (w - hi.astype(jnp.float32)).astype(jnp.bfloat16)
    return hi, lo


def _mlp_body(u_ref, m_ref, w1uh_ref, w1ul_ref, w1mh_ref, w1ml_ref, b1_ref,
              w2h_ref, w2l_ref, b2_ref, w3h_ref, w3l_ref, b3_ref,
              wf_ref, bf_ref, out_ref):
    f32 = jnp.float32

    def dot(x, wh, wl):
        xh = x.astype(jnp.bfloat16)
        xl = (x - xh.astype(f32)).astype(jnp.bfloat16)
        acc = jnp.dot(xh, wh, preferred_element_type=f32)
        acc += jnp.dot(xh, wl, preferred_element_type=f32)
        acc += jnp.dot(xl, wh, preferred_element_type=f32)
        return acc

    x = (dot(u_ref[...], w1uh_ref[...], w1ul_ref[...])
         + dot(m_ref[...], w1mh_ref[...], w1ml_ref[...]) + b1_ref[...])
    x = jnp.maximum(x, 0.0)
    x = jnp.maximum(dot(x, w2h_ref[...], w2l_ref[...]) + b2_ref[...], 0.0)
    x = jnp.maximum(dot(x, w3h_ref[...], w3l_ref[...]) + b3_ref[...], 0.0)
    z = jnp.sum(x * wf_ref[...], axis=1, keepdims=True) + bf_ref[...]
    out_ref[...] = jax.nn.sigmoid(z) * 4.0 + 0.5


def _mlp(u_emb, m_emb, w1uh, w1ul, w1mh, w1ml, b1, w2h, w2l, b2,
         w3h, w3l, b3, wf, bf):
    full = lambda shape: pl.BlockSpec(shape, lambda i: (0,) * len(shape))
    return pl.pallas_call(
        _mlp_body,
        grid=(B // BB,),
        in_specs=[
            pl.BlockSpec((BB, D), lambda i: (i, 0)),
            pl.BlockSpec((BB, D), lambda i: (i, 0)),
            full((D, 128)), full((D, 128)), full((D, 128)), full((D, 128)),
            full((1, 128)),
            full((128, 256)), full((128, 256)), full((1, 256)),
            full((256, 128)), full((256, 128)), full((1, 128)),
            full((1, 128)), full((1, 1)),
        ],
        out_specs=pl.BlockSpec((BB, 1), lambda i: (i, 0)),
        out_shape=jax.ShapeDtypeStruct((B, 1), jnp.float32),
    )(u_emb, m_emb, w1uh, w1ul, w1mh, w1ml, b1, w2h, w2l, b2,
      w3h, w3l, b3, wf, bf)


@jax.jit
def kernel(users, movies, Uw, Mw, W1, b1, W2, b2, W3, b3, Wf, bf):
    u_emb, m_emb = _make_gather()(users.astype(jnp.int32),
                                  movies.astype(jnp.int32), Uw, Mw)
    w1uh, w1ul = _split(W1[:D])
    w1mh, w1ml = _split(W1[D:])
    w2h, w2l = _split(W2)
    w3h, w3l = _split(W3)
    return _mlp(u_emb, m_emb, w1uh, w1ul, w1mh, w1ml, b1[None, :],
                w2h, w2l, b2[None, :], w3h, w3l, b3[None, :],
                Wf.reshape(1, 128), bf[None, :])


# speedup vs baseline: 1.4573x; 1.0205x over previous
import functools

import jax
import jax.numpy as jnp
from jax import lax
from jax.experimental import pallas as pl
from jax.experimental.pallas import tpu as pltpu
from jax.experimental.pallas import tpu_sc as plsc

B = 16384
D = 50
NC = 2
NS = 16
NW = NC * NS
BPW = B // NW
ICHUNK = 128
NCHUNK = BPW // ICHUNK


def _gather_body(users_hbm, movies_hbm, uw_hbm, mw_hbm, uout_hbm, mout_hbm,
                 uidx, midx, urows, mrows, usem0, usem1, msem0, msem1):
    usems = (usem0, usem1)
    msems = (msem0, msem1)
    wid = lax.axis_index("s") * NC + lax.axis_index("c")
    base = wid * BPW
    pltpu.sync_copy(users_hbm.at[pl.ds(base, BPW)], uidx)
    pltpu.sync_copy(movies_hbm.at[pl.ds(base, BPW)], midx)

    L = 16

    def issue(c, slot):
        off = c * ICHUNK

        def issue_g(g, _):
            uvec = uidx[pl.ds(off + g * L, L)]
            mvec = midx[pl.ds(off + g * L, L)]
            for l in range(L):
                pltpu.async_copy(uw_hbm.at[uvec[l]],
                                 urows.at[slot, g * L + l], usems[slot])
                pltpu.async_copy(mw_hbm.at[mvec[l]],
                                 mrows.at[slot, g * L + l], msems[slot])
            return 0

        lax.fori_loop(0, ICHUNK // L, issue_g, 0)

    def drain_and_store(c, slot):
        def drain(i, _):
            pltpu.make_async_copy(uw_hbm.at[0], urows.at[0, 0],
                                  usems[slot]).wait()
            pltpu.make_async_copy(mw_hbm.at[0], mrows.at[0, 0],
                                  msems[slot]).wait()
            return 0

        lax.fori_loop(0, ICHUNK, drain, 0)
        off = c * ICHUNK
        pltpu.sync_copy(urows.at[slot], uout_hbm.at[pl.ds(base + off, ICHUNK)])
        pltpu.sync_copy(mrows.at[slot], mout_hbm.at[pl.ds(base + off, ICHUNK)])

    issue(0, 0)
    for c in range(NCHUNK - 1):
        issue(c + 1, (c + 1) % 2)
        drain_and_store(c, c % 2)
    drain_and_store(NCHUNK - 1, (NCHUNK - 1) % 2)


@functools.cache
def _make_gather():
    return pl.kernel(
        _gather_body,
        out_type=(jax.ShapeDtypeStruct((B, D), jnp.float32),
                  jax.ShapeDtypeStruct((B, D), jnp.float32)),
        mesh=plsc.VectorSubcoreMesh(core_axis_name="c", subcore_axis_name="s",
                                    num_cores=NC, num_subcores=NS),
        scratch_types=[
            pltpu.VMEM((BPW,), jnp.int32),
            pltpu.VMEM((BPW,), jnp.int32),
            pltpu.VMEM((2, ICHUNK, D), jnp.float32),
            pltpu.VMEM((2, ICHUNK, D), jnp.float32),
            pltpu.SemaphoreType.DMA,
            pltpu.SemaphoreType.DMA,
            pltpu.SemaphoreType.DMA,
            pltpu.SemaphoreType.DMA,
        ],
    )


BB = 2048


def _split(w):
    hi = w.astype(jnp.bfloat16)
    lo = (w - hi.astype(jnp.float32)).astype(jnp.bfloat16)
    return hi, lo


def _mlp_body(u_ref, m_ref, w1uh_ref, w1ul_ref, w1mh_ref, w1ml_ref, b1_ref,
              w2h_ref, w2l_ref, b2_ref, w3h_ref, w3l_ref, b3_ref,
              wf_ref, bf_ref, out_ref):
    f32 = jnp.float32

    def dot(x, wh, wl):
        xh = x.astype(jnp.bfloat16)
        xl = (x - xh.astype(f32)).astype(jnp.bfloat16)
        acc = jnp.dot(xh, wh, preferred_element_type=f32)
        acc += jnp.dot(xh, wl, preferred_element_type=f32)
        acc += jnp.dot(xl, wh, preferred_element_type=f32)
        return acc

    x = (dot(u_ref[...], w1uh_ref[...], w1ul_ref[...])
         + dot(m_ref[...], w1mh_ref[...], w1ml_ref[...]) + b1_ref[...])
    x = jnp.maximum(x, 0.0)
    x = jnp.maximum(dot(x, w2h_ref[...], w2l_ref[...]) + b2_ref[...], 0.0)
    x = jnp.maximum(dot(x, w3h_ref[...], w3l_ref[...]) + b3_ref[...], 0.0)
    z = jnp.sum(x * wf_ref[...], axis=1, keepdims=True) + bf_ref[...]
    out_ref[...] = jax.nn.sigmoid(z) * 4.0 + 0.5


def _mlp(u_emb, m_emb, w1uh, w1ul, w1mh, w1ml, b1, w2h, w2l, b2,
         w3h, w3l, b3, wf, bf):
    full = lambda shape: pl.BlockSpec(shape, lambda i: (0,) * len(shape))
    return pl.pallas_call(
        _mlp_body,
        grid=(B // BB,),
        in_specs=[
            pl.BlockSpec((BB, D), lambda i: (i, 0)),
            pl.BlockSpec((BB, D), lambda i: (i, 0)),
            full((D, 128)), full((D, 128)), full((D, 128)), full((D, 128)),
            full((1, 128)),
            full((128, 256)), full((128, 256)), full((1, 256)),
            full((256, 128)), full((256, 128)), full((1, 128)),
            full((1, 128)), full((1, 1)),
        ],
        out_specs=pl.BlockSpec((BB, 1), lambda i: (i, 0)),
        out_shape=jax.ShapeDtypeStruct((B, 1), jnp.float32),
    )(u_emb, m_emb, w1uh, w1ul, w1mh, w1ml, b1, w2h, w2l, b2,
      w3h, w3l, b3, wf, bf)


@jax.jit
def kernel(users, movies, Uw, Mw, W1, b1, W2, b2, W3, b3, Wf, bf):
    u_emb, m_emb = _make_gather()(users.astype(jnp.int32),
                                  movies.astype(jnp.int32), Uw, Mw)
    w1uh, w1ul = _split(W1[:D])
    w1mh, w1ml = _split(W1[D:])
    w2h, w2l = _split(W2)
    w3h, w3l = _split(W3)
    return _mlp(u_emb, m_emb, w1uh, w1ul, w1mh, w1ml, b1[None, :],
                w2h, w2l, b2[None, :], w3h, w3l, b3[None, :],
                Wf.reshape(1, 128), bf[None, :])


# speedup vs baseline: 1.4692x; 1.0081x over previous
import functools

import jax
import jax.numpy as jnp
from jax import lax
from jax.experimental import pallas as pl
from jax.experimental.pallas import tpu as pltpu
from jax.experimental.pallas import tpu_sc as plsc

B = 16384
D = 50
NC = 2
NS = 16
NW = NC * NS
BPW = B // NW
ICHUNK = 128
NCHUNK = BPW // ICHUNK


def _gather_body(users_hbm, movies_hbm, uw_hbm, mw_hbm, uout_hbm, mout_hbm,
                 uidx, midx, urows, mrows, usem0, usem1, msem0, msem1):
    usems = (usem0, usem1)
    msems = (msem0, msem1)
    wid = lax.axis_index("s") * NC + lax.axis_index("c")
    base = wid * BPW
    pltpu.sync_copy(users_hbm.at[pl.ds(base, BPW)], uidx)
    pltpu.sync_copy(movies_hbm.at[pl.ds(base, BPW)], midx)

    L = 16

    def issue(c, slot):
        off = c * ICHUNK

        def issue_g(g, _):
            uvec = uidx[pl.ds(off + g * L, L)]
            mvec = midx[pl.ds(off + g * L, L)]
            for l in range(L):
                pltpu.async_copy(uw_hbm.at[uvec[l]],
                                 urows.at[slot, g * L + l], usems[slot])
                pltpu.async_copy(mw_hbm.at[mvec[l]],
                                 mrows.at[slot, g * L + l], msems[slot])
            return 0

        lax.fori_loop(0, ICHUNK // L, issue_g, 0)

    def drain_and_store(c, slot):
        def drain(i, _):
            pltpu.make_async_copy(uw_hbm.at[0], urows.at[0, 0],
                                  usems[slot]).wait()
            pltpu.make_async_copy(mw_hbm.at[0], mrows.at[0, 0],
                                  msems[slot]).wait()
            return 0

        lax.fori_loop(0, ICHUNK, drain, 0)
        off = c * ICHUNK
        pltpu.sync_copy(urows.at[slot], uout_hbm.at[pl.ds(base + off, ICHUNK)])
        pltpu.sync_copy(mrows.at[slot], mout_hbm.at[pl.ds(base + off, ICHUNK)])

    issue(0, 0)
    for c in range(NCHUNK - 1):
        issue(c + 1, (c + 1) % 2)
        drain_and_store(c, c % 2)
    drain_and_store(NCHUNK - 1, (NCHUNK - 1) % 2)


@functools.cache
def _make_gather():
    return pl.kernel(
        _gather_body,
        out_type=(jax.ShapeDtypeStruct((B, D), jnp.float32),
                  jax.ShapeDtypeStruct((B, D), jnp.float32)),
        mesh=plsc.VectorSubcoreMesh(core_axis_name="c", subcore_axis_name="s",
                                    num_cores=NC, num_subcores=NS),
        scratch_types=[
            pltpu.VMEM((BPW,), jnp.int32),
            pltpu.VMEM((BPW,), jnp.int32),
            pltpu.VMEM((2, ICHUNK, D), jnp.float32),
            pltpu.VMEM((2, ICHUNK, D), jnp.float32),
            pltpu.SemaphoreType.DMA,
            pltpu.SemaphoreType.DMA,
            pltpu.SemaphoreType.DMA,
            pltpu.SemaphoreType.DMA,
        ],
    )


BB = 4096


def _split(w):
    hi = w.astype(jnp.bfloat16)
    lo = (w - hi.astype(jnp.float32)).astype(jnp.bfloat16)
    return hi, lo


def _mlp_body(u_ref, m_ref, w1uh_ref, w1ul_ref, w1mh_ref, w1ml_ref, b1_ref,
              w2h_ref, w2l_ref, b2_ref, w3h_ref, w3l_ref, b3_ref,
              wf_ref, bf_ref, out_ref):
    f32 = jnp.float32

    def dot(x, wh, wl):
        xh = x.astype(jnp.bfloat16)
        xl = (x - xh.astype(f32)).astype(jnp.bfloat16)
        acc = jnp.dot(xh, wh, preferred_element_type=f32)
        acc += jnp.dot(xh, wl, preferred_element_type=f32)
        acc += jnp.dot(xl, wh, preferred_element_type=f32)
        return acc

    x = (dot(u_ref[...], w1uh_ref[...], w1ul_ref[...])
         + dot(m_ref[...], w1mh_ref[...], w1ml_ref[...]) + b1_ref[...])
    x = jnp.maximum(x, 0.0)
    x = jnp.maximum(dot(x, w2h_ref[...], w2l_ref[...]) + b2_ref[...], 0.0)
    x = jnp.maximum(dot(x, w3h_ref[...], w3l_ref[...]) + b3_ref[...], 0.0)
    z = jnp.sum(x * wf_ref[...], axis=1, keepdims=True) + bf_ref[...]
    out_ref[...] = jax.nn.sigmoid(z) * 4.0 + 0.5


def _mlp(u_emb, m_emb, w1uh, w1ul, w1mh, w1ml, b1, w2h, w2l, b2,
         w3h, w3l, b3, wf, bf):
    full = lambda shape: pl.BlockSpec(shape, lambda i: (0,) * len(shape))
    return pl.pallas_call(
        _mlp_body,
        grid=(B // BB,),
        in_specs=[
            pl.BlockSpec((BB, D), lambda i: (i, 0)),
            pl.BlockSpec((BB, D), lambda i: (i, 0)),
            full((D, 128)), full((D, 128)), full((D, 128)), full((D, 128)),
            full((1, 128)),
            full((128, 256)), full((128, 256)), full((1, 256)),
            full((256, 128)), full((256, 128)), full((1, 128)),
            full((1, 128)), full((1, 1)),
        ],
        out_specs=pl.BlockSpec((BB, 1), lambda i: (i, 0)),
        out_shape=jax.ShapeDtypeStruct((B, 1), jnp.float32),
    )(u_emb, m_emb, w1uh, w1ul, w1mh, w1ml, b1, w2h, w2l, b2,
      w3h, w3l, b3, wf, bf)


@jax.jit
def kernel(users, movies, Uw, Mw, W1, b1, W2, b2, W3, b3, Wf, bf):
    u_emb, m_emb = _make_gather()(users.astype(jnp.int32),
                                  movies.astype(jnp.int32), Uw, Mw)
    w1uh, w1ul = _split(W1[:D])
    w1mh, w1ml = _split(W1[D:])
    w2h, w2l = _split(W2)
    w3h, w3l = _split(W3)
    return _mlp(u_emb, m_emb, w1uh, w1ul, w1mh, w1ml, b1[None, :],
                w2h, w2l, b2[None, :], w3h, w3l, b3[None, :],
                Wf.reshape(1, 128), bf[None, :])
